# baseline (speedup 1.0000x reference)
import jax
import jax.numpy as jnp
from jax import lax
from jax.experimental import pallas as pl
from jax.experimental.pallas import tpu as pltpu
from jax.experimental.pallas import tpu_sc as plsc

L = 16
V = 8192
NROWS = 512
NWORKERS = 32
ROWS_PER = NROWS // NWORKERS
CH = 1
NCHUNKS = ROWS_PER // CH
NCHAIN = 4
NSTEP = V // (L * NCHAIN)


def _merge(ma, ia, mb, ib):
    take = (mb > ma) | ((mb == ma) & (ib < ia))
    return jnp.where(take, mb, ma), jnp.where(take, ib, ia)


def _argmax_row(xbuf, off, lanes):
    ms = [jnp.full((L,), -jnp.inf, jnp.float32) for _ in range(NCHAIN)]
    iis = [jnp.zeros((L,), jnp.int32) for _ in range(NCHAIN)]
    curs = [lanes + L * k for k in range(NCHAIN)]

    def cbody(j, carry):
        ms, iis, curs = carry
        base = off + j * (L * NCHAIN)
        nms, nis, ncurs = [], [], []
        for k in range(NCHAIN):
            x = xbuf[pl.ds(base + k * L, L)]
            cond = x > ms[k]
            nms.append(jnp.where(cond, x, ms[k]))
            nis.append(jnp.where(cond, curs[k], iis[k]))
            ncurs.append(curs[k] + L * NCHAIN)
        return (tuple(nms), tuple(nis), tuple(ncurs))

    ms, iis, _ = lax.fori_loop(
        0, NSTEP, cbody, (tuple(ms), tuple(iis), tuple(curs))
    )

    m01, i01 = _merge(ms[0], iis[0], ms[1], iis[1])
    m23, i23 = _merge(ms[2], iis[2], ms[3], iis[3])
    m, idx = _merge(m01, i01, m23, i23)

    gm = m[0]
    gi = idx[0]
    for k in range(1, L):
        mv = m[k]
        iv = idx[k]
        take = (mv > gm) | ((mv == gm) & (iv < gi))
        gm = jnp.where(take, mv, gm)
        gi = jnp.where(take, iv, gi)
    return gi


def _body(x_hbm, out_hbm, xb0, xb1, ob0, ob1, si0, si1, so0, so1):
    cid = lax.axis_index("c")
    sid = lax.axis_index("s")
    wid = sid * 2 + cid
    base = wid * ROWS_PER * V

    xbufs = (xb0, xb1)
    obufs = (ob0, ob1)
    sins = (si0, si1)
    souts = (so0, so1)

    lanes = lax.iota(jnp.int32, L)
    zeros = jnp.zeros((L,), jnp.float32)
    ones = jnp.ones((L,), jnp.float32)
    mask0 = lanes == 0

    def zbody(j, c):
        ob0[pl.ds(j * L, L)] = zeros
        ob1[pl.ds(j * L, L)] = zeros
        return c

    lax.fori_loop(0, CH * V // L, zbody, 0)

    pltpu.async_copy(x_hbm.at[pl.ds(base, CH * V)], xb0, si0)

    prev = [None, None]
    for c in range(NCHUNKS):
        slot = c % 2
        start = base + c * CH * V
        pltpu.make_async_copy(
            x_hbm.at[pl.ds(start, CH * V)], xbufs[slot], sins[slot]
        ).wait()
        if c + 1 < NCHUNKS:
            nstart = base + (c + 1) * CH * V
            pltpu.async_copy(
                x_hbm.at[pl.ds(nstart, CH * V)], xbufs[1 - slot], sins[1 - slot]
            )

        idxvs = []
        for r in range(CH):
            gi = _argmax_row(xbufs[slot], r * V, lanes)
            idxvs.append(jnp.full((L,), gi + r * V, jnp.int32))

        if c >= 2:
            pstart = base + (c - 2) * CH * V
            pltpu.make_async_copy(
                obufs[slot], out_hbm.at[pl.ds(pstart, CH * V)], souts[slot]
            ).wait()
            for r in range(CH):
                plsc.store_scatter(obufs[slot], [prev[slot][r]], zeros, mask=mask0)

        for r in range(CH):
            plsc.store_scatter(obufs[slot], [idxvs[r]], ones, mask=mask0)
        pltpu.async_copy(obufs[slot], out_hbm.at[pl.ds(start, CH * V)], souts[slot])
        prev[slot] = idxvs

    pltpu.make_async_copy(
        ob0, out_hbm.at[pl.ds(base + (NCHUNKS - 2) * CH * V, CH * V)], so0
    ).wait()
    pltpu.make_async_copy(
        ob1, out_hbm.at[pl.ds(base + (NCHUNKS - 1) * CH * V, CH * V)], so1
    ).wait()


@jax.jit
def kernel(logits):
    B, S, _ = logits.shape
    x = logits.reshape(NROWS * V)
    out = pl.kernel(
        _body,
        out_type=jax.ShapeDtypeStruct((NROWS * V,), jnp.float32),
        mesh=plsc.VectorSubcoreMesh(core_axis_name="c", subcore_axis_name="s"),
        compiler_params=pltpu.CompilerParams(needs_layout_passes=False),
        scratch_types=[
            pltpu.VMEM((CH * V,), jnp.float32),
            pltpu.VMEM((CH * V,), jnp.float32),
            pltpu.VMEM((CH * V,), jnp.float32),
            pltpu.VMEM((CH * V,), jnp.float32),
            pltpu.SemaphoreType.DMA,
            pltpu.SemaphoreType.DMA,
            pltpu.SemaphoreType.DMA,
            pltpu.SemaphoreType.DMA,
        ],
    )(x)
    return out.reshape(B, S, V)


# speedup vs baseline: 2.0460x; 2.0460x over previous
import jax
import jax.numpy as jnp
from jax import lax
from jax.experimental import pallas as pl
from jax.experimental.pallas import tpu as pltpu
from jax.experimental.pallas import tpu_sc as plsc

L = 16
V = 8192
NROWS = 512
NWORKERS = 32
ROWS_PER = NROWS // NWORKERS
CH = 2
NCHUNKS = ROWS_PER // CH
NCHAIN = 4
NSTEP = V // (L * NCHAIN)


def _merge(ma, ia, mb, ib):
    take = (mb > ma) | ((mb == ma) & (ib < ia))
    return jnp.where(take, mb, ma), jnp.where(take, ib, ia)


def _argmax_row(xbuf, r, lanes):
    ms = [jnp.full((L,), -jnp.inf, jnp.float32) for _ in range(NCHAIN)]
    iis = [jnp.zeros((L,), jnp.int32) for _ in range(NCHAIN)]
    curs = [lanes + L * k for k in range(NCHAIN)]

    def cbody(j, carry):
        ms, iis, curs = carry
        base = j * (L * NCHAIN)
        nms, nis, ncurs = [], [], []
        for k in range(NCHAIN):
            x = xbuf[r, pl.ds(base + k * L, L)]
            cond = x > ms[k]
            nms.append(jnp.where(cond, x, ms[k]))
            nis.append(jnp.where(cond, curs[k], iis[k]))
            ncurs.append(curs[k] + L * NCHAIN)
        return (tuple(nms), tuple(nis), tuple(ncurs))

    ms, iis, _ = lax.fori_loop(
        0, NSTEP, cbody, (tuple(ms), tuple(iis), tuple(curs))
    )

    m01, i01 = _merge(ms[0], iis[0], ms[1], iis[1])
    m23, i23 = _merge(ms[2], iis[2], ms[3], iis[3])
    m, idx = _merge(m01, i01, m23, i23)

    gm = m[0]
    gi = idx[0]
    for k in range(1, L):
        mv = m[k]
        iv = idx[k]
        take = (mv > gm) | ((mv == gm) & (iv < gi))
        gm = jnp.where(take, mv, gm)
        gi = jnp.where(take, iv, gi)
    return gi


def _body(x_hbm, out_hbm, xb0, xb1, ob0, ob1, si0, si1, so0, so1):
    cid = lax.axis_index("c")
    sid = lax.axis_index("s")
    wid = sid * 2 + cid
    base = wid * ROWS_PER

    xbufs = (xb0, xb1)
    obufs = (ob0, ob1)
    sins = (si0, si1)
    souts = (so0, so1)

    lanes = lax.iota(jnp.int32, L)
    zeros = jnp.zeros((L,), jnp.float32)
    ones = jnp.ones((L,), jnp.float32)
    mask0 = lanes == 0

    def zbody(j, c):
        for r in range(CH):
            ob0[r, pl.ds(j * L, L)] = zeros
            ob1[r, pl.ds(j * L, L)] = zeros
        return c

    lax.fori_loop(0, V // L, zbody, 0)

    pltpu.async_copy(x_hbm.at[pl.ds(base, CH)], xb0, si0)

    prev = [None, None]
    for c in range(NCHUNKS):
        slot = c % 2
        row = base + c * CH
        pltpu.make_async_copy(
            x_hbm.at[pl.ds(row, CH)], xbufs[slot], sins[slot]
        ).wait()
        if c + 1 < NCHUNKS:
            pltpu.async_copy(
                x_hbm.at[pl.ds(row + CH, CH)], xbufs[1 - slot], sins[1 - slot]
            )

        idxvs = []
        for r in range(CH):
            gi = _argmax_row(xbufs[slot], r, lanes)
            idxvs.append((jnp.full((L,), r, jnp.int32), jnp.full((L,), gi, jnp.int32)))

        if c >= 2:
            prow = base + (c - 2) * CH
            pltpu.make_async_copy(
                obufs[slot], out_hbm.at[pl.ds(prow, CH)], souts[slot]
            ).wait()
            for r in range(CH):
                plsc.store_scatter(
                    obufs[slot], list(prev[slot][r]), zeros, mask=mask0
                )

        for r in range(CH):
            plsc.store_scatter(obufs[slot], list(idxvs[r]), ones, mask=mask0)
        pltpu.async_copy(obufs[slot], out_hbm.at[pl.ds(row, CH)], souts[slot])
        prev[slot] = idxvs

    pltpu.make_async_copy(
        ob0, out_hbm.at[pl.ds(base + (NCHUNKS - 2) * CH, CH)], so0
    ).wait()
    pltpu.make_async_copy(
        ob1, out_hbm.at[pl.ds(base + (NCHUNKS - 1) * CH, CH)], so1
    ).wait()


@jax.jit
def kernel(logits):
    B, S, _ = logits.shape
    x = logits.reshape(NROWS, V)
    out = pl.kernel(
        _body,
        out_type=jax.ShapeDtypeStruct((NROWS, V), jnp.float32),
        mesh=plsc.VectorSubcoreMesh(core_axis_name="c", subcore_axis_name="s"),
        compiler_params=pltpu.CompilerParams(needs_layout_passes=False),
        scratch_types=[
            pltpu.VMEM((CH, V), jnp.float32),
            pltpu.VMEM((CH, V), jnp.float32),
            pltpu.VMEM((CH, V), jnp.float32),
            pltpu.VMEM((CH, V), jnp.float32),
            pltpu.SemaphoreType.DMA,
            pltpu.SemaphoreType.DMA,
            pltpu.SemaphoreType.DMA,
            pltpu.SemaphoreType.DMA,
        ],
    )(x)
    return out.reshape(B, S, V)
